# baseline (speedup 1.0000x reference)
import functools

import jax
import jax.numpy as jnp
from jax.experimental import pallas as pl
from jax.experimental.pallas import tpu as pltpu

MM = 1024
HID = 2048
N_Q = 576
HEADS = 8
HEAD_DIM = 128
BATCH = 4
TOKENS = BATCH * N_Q

def _bf(a):
    return a.astype(jnp.bfloat16)


def _DOT(a, b):
    return jnp.dot(_bf(a), _bf(b), preferred_element_type=jnp.float32)


def _dot_t(a, b):
    return jax.lax.dot_general(_bf(a), _bf(b), (((1,), (1,)), ((), ())),
                               preferred_element_type=jnp.float32)


def _ln(x, g, b, eps=1e-5):
    mu = jnp.mean(x, axis=-1, keepdims=True)
    var = jnp.mean((x - mu) ** 2, axis=-1, keepdims=True)
    return (x - mu) * jax.lax.rsqrt(var + eps) * g + b


def _gelu(h):
    return 0.5 * h * (1.0 + jax.lax.erf(h * 0.7071067811865476))


def _q_kernel(query_ref, pos_ref, g_ref, b_ref, wq_ref, bq_ref, qh_ref):
    qf = _ln(query_ref[...], g_ref[...], b_ref[...]) + pos_ref[...]
    qh_ref[...] = _dot_t(qf, wq_ref[...]) + bq_ref[...]


def _resampler_kernel(qh_ref, x_ref, pos_ref, g_ref, b_ref, wk_ref, wv_ref,
                      bk_ref, bv_ref, ow_ref, ob_ref, o_ref):
    kv = _ln(x_ref[...], g_ref[...], b_ref[...])
    kh = _dot_t(kv + pos_ref[...], wk_ref[...]) + bk_ref[...]
    vh = _dot_t(kv, wv_ref[...]) + bv_ref[...]
    qh = qh_ref[...]
    scale = 1.0 / (HEAD_DIM ** 0.5)
    outs = []
    for h in range(HEADS):
        sl = slice(h * HEAD_DIM, (h + 1) * HEAD_DIM)
        s = _dot_t(qh[:, sl], kh[:, sl]) * scale
        s = s - jnp.max(s, axis=-1, keepdims=True)
        e = jnp.exp(s)
        p = e / jnp.sum(e, axis=-1, keepdims=True)
        outs.append(_DOT(p, vh[:, sl]))
    o = jnp.concatenate(outs, axis=1)
    o_ref[...] = _bf(_dot_t(o, ow_ref[...]) + ob_ref[...])


def _h_kernel(x_ref, a_ref, w1_ref, b1_ref, wgt_ref, hm_ref, g_ref):
    x = x_ref[...]
    wgt = wgt_ref[...]
    l0 = jnp.sum(x * wgt[0:1, :], axis=1, keepdims=True)
    l1 = jnp.sum(x * wgt[1:2, :], axis=1, keepdims=True)
    m = jnp.maximum(l0, l1)
    e0 = jnp.exp(l0 - m)
    e1 = jnp.exp(l1 - m)
    s = e0 + e1
    p0 = e0 / s
    p1 = e1 / s
    denom = p0 + p1 + 1e-6
    g0 = p0 / denom
    g1 = p1 / denom
    h0 = _gelu(_DOT(x, w1_ref[...]) + b1_ref[...])
    h1 = _gelu(_DOT(a_ref[...], w1_ref[...]) + b1_ref[...])
    hm_ref[...] = _bf(g0 * h0 + g1 * h1)
    g_ref[...] = jnp.concatenate([g0, g1], axis=1)


def _out_kernel(hm_ref, w2_ref, b2_ref, g_ref, o_ref):
    g = g_ref[...]
    gsum = g[:, 0:1] + g[:, 1:2]
    o_ref[...] = _DOT(hm_ref[...], w2_ref[...]) + gsum * b2_ref[...]


def kernel(x, proj_w1, proj_b1, proj_w2, proj_b2, query, ln_q_g, ln_q_b,
           ln_kv_g, ln_kv_b, in_proj_w, in_proj_b, out_proj_w, out_proj_b,
           w_gate, pos_embed):
    f32 = jnp.float32
    xf = x.reshape(TOKENS, MM)
    bq, bk, bv = (in_proj_b[:MM][None, :], in_proj_b[MM:2 * MM][None, :],
                  in_proj_b[2 * MM:][None, :])
    ln_q_g2, ln_q_b2 = ln_q_g[None, :], ln_q_b[None, :]
    ln_kv_g2, ln_kv_b2 = ln_kv_g[None, :], ln_kv_b[None, :]
    b1_2 = proj_b1[None, :]
    b2_2 = proj_b2[None, :]
    ob_2 = out_proj_b[None, :]
    wgt = w_gate.T
    bf16 = jnp.bfloat16
    ipw_bf = in_proj_w.astype(bf16)
    ow_bf = out_proj_w.astype(bf16)
    w1_bf = proj_w1.astype(bf16)
    w2_bf = proj_w2.astype(bf16)

    qh = pl.pallas_call(
        _q_kernel,
        grid=(1,),
        in_specs=[
            pl.BlockSpec((N_Q, MM), lambda i: (0, 0)),
            pl.BlockSpec((N_Q, MM), lambda i: (0, 0)),
            pl.BlockSpec((1, MM), lambda i: (0, 0)),
            pl.BlockSpec((1, MM), lambda i: (0, 0)),
            pl.BlockSpec((MM, MM), lambda i: (0, 0)),
            pl.BlockSpec((1, MM), lambda i: (0, 0)),
        ],
        out_specs=pl.BlockSpec((N_Q, MM), lambda i: (0, 0)),
        out_shape=jax.ShapeDtypeStruct((N_Q, MM), f32),
    )(query, pos_embed, ln_q_g2, ln_q_b2, ipw_bf, bq)

    attn_out = pl.pallas_call(
        _resampler_kernel,
        grid=(BATCH,),
        in_specs=[
            pl.BlockSpec((N_Q, MM), lambda i: (0, 0)),
            pl.BlockSpec((N_Q, MM), lambda i: (i, 0)),
            pl.BlockSpec((N_Q, MM), lambda i: (0, 0)),
            pl.BlockSpec((1, MM), lambda i: (0, 0)),
            pl.BlockSpec((1, MM), lambda i: (0, 0)),
            pl.BlockSpec((MM, MM), lambda i: (1, 0)),
            pl.BlockSpec((MM, MM), lambda i: (2, 0)),
            pl.BlockSpec((1, MM), lambda i: (0, 0)),
            pl.BlockSpec((1, MM), lambda i: (0, 0)),
            pl.BlockSpec((MM, MM), lambda i: (0, 0)),
            pl.BlockSpec((1, MM), lambda i: (0, 0)),
        ],
        out_specs=pl.BlockSpec((N_Q, MM), lambda i: (i, 0)),
        out_shape=jax.ShapeDtypeStruct((TOKENS, MM), bf16),
        compiler_params=pltpu.CompilerParams(
            dimension_semantics=("parallel",)),
    )(qh, xf, pos_embed, ln_kv_g2, ln_kv_b2, ipw_bf, ipw_bf,
      bk, bv, ow_bf, ob_2)

    MT = 256
    hm, gates = pl.pallas_call(
        _h_kernel,
        grid=(TOKENS // MT,),
        in_specs=[
            pl.BlockSpec((MT, MM), lambda i: (i, 0)),
            pl.BlockSpec((MT, MM), lambda i: (i, 0)),
            pl.BlockSpec((MM, HID), lambda i: (0, 0)),
            pl.BlockSpec((1, HID), lambda i: (0, 0)),
            pl.BlockSpec((2, MM), lambda i: (0, 0)),
        ],
        out_specs=[
            pl.BlockSpec((MT, HID), lambda i: (i, 0)),
            pl.BlockSpec((MT, 2), lambda i: (i, 0)),
        ],
        out_shape=[
            jax.ShapeDtypeStruct((TOKENS, HID), bf16),
            jax.ShapeDtypeStruct((TOKENS, 2), f32),
        ],
        compiler_params=pltpu.CompilerParams(
            dimension_semantics=("parallel",)),
    )(xf, attn_out, w1_bf, b1_2, wgt)

    out = pl.pallas_call(
        _out_kernel,
        grid=(TOKENS // MT,),
        in_specs=[
            pl.BlockSpec((MT, HID), lambda i: (i, 0)),
            pl.BlockSpec((HID, HID), lambda i: (0, 0)),
            pl.BlockSpec((1, HID), lambda i: (0, 0)),
            pl.BlockSpec((MT, 2), lambda i: (i, 0)),
        ],
        out_specs=pl.BlockSpec((MT, HID), lambda i: (i, 0)),
        out_shape=jax.ShapeDtypeStruct((TOKENS, HID), f32),
        compiler_params=pltpu.CompilerParams(
            dimension_semantics=("parallel",)),
    )(hm, w2_bf, b2_2, gates)

    return out.reshape(BATCH, N_Q, HID)


# speedup vs baseline: 1.1822x; 1.1822x over previous
import functools

import jax
import jax.numpy as jnp
from jax.experimental import pallas as pl
from jax.experimental.pallas import tpu as pltpu

MM = 1024
HID = 2048
N_Q = 576
HEADS = 8
HEAD_DIM = 128
BATCH = 4
TOKENS = BATCH * N_Q

def _bf(a):
    return a.astype(jnp.bfloat16)


def _DOT(a, b):
    return jnp.dot(a, b, preferred_element_type=jnp.float32)


def _dot_t(a, b):
    return jax.lax.dot_general(a, b, (((1,), (1,)), ((), ())),
                               preferred_element_type=jnp.float32)


def _ln(x, g, b, eps=1e-5):
    mu = jnp.mean(x, axis=-1, keepdims=True)
    var = jnp.mean((x - mu) ** 2, axis=-1, keepdims=True)
    return (x - mu) * jax.lax.rsqrt(var + eps) * g + b


def _gelu(h):
    return 0.5 * h * (1.0 + jax.lax.erf(h * 0.7071067811865476))


def _q_kernel(query_ref, pos_ref, g_ref, b_ref, wq_ref, bq_ref, qh_ref):
    qf = _ln(query_ref[...], g_ref[...], b_ref[...]) + pos_ref[...]
    qh_ref[...] = _dot_t(qf, wq_ref[...]) + bq_ref[...]


def _resampler_kernel(qh_ref, x_ref, pos_ref, g_ref, b_ref, wk_ref, wv_ref,
                      bk_ref, bv_ref, ow_ref, ob_ref, o_ref):
    kv = _ln(x_ref[...], g_ref[...], b_ref[...])
    kh = _dot_t(kv + pos_ref[...], wk_ref[...]) + bk_ref[...]
    vh = _dot_t(kv, wv_ref[...]) + bv_ref[...]
    qh = qh_ref[...]
    scale = 1.0 / (HEAD_DIM ** 0.5)
    outs = []
    for h in range(HEADS):
        sl = slice(h * HEAD_DIM, (h + 1) * HEAD_DIM)
        s = _dot_t(qh[:, sl], kh[:, sl]) * scale
        s = s - jnp.max(s, axis=-1, keepdims=True)
        e = jnp.exp(s)
        p = e / jnp.sum(e, axis=-1, keepdims=True)
        outs.append(_DOT(p, vh[:, sl]))
    o = jnp.concatenate(outs, axis=1)
    o_ref[...] = _bf(_dot_t(o, ow_ref[...]) + ob_ref[...])


def _h_kernel(x_ref, a_ref, w1_ref, b1_ref, wgt_ref, hm_ref, g_ref):
    x = x_ref[...]
    wgt = wgt_ref[...]
    l0 = jnp.sum(x * wgt[0:1, :], axis=1, keepdims=True)
    l1 = jnp.sum(x * wgt[1:2, :], axis=1, keepdims=True)
    m = jnp.maximum(l0, l1)
    e0 = jnp.exp(l0 - m)
    e1 = jnp.exp(l1 - m)
    s = e0 + e1
    p0 = e0 / s
    p1 = e1 / s
    denom = p0 + p1 + 1e-6
    g0 = p0 / denom
    g1 = p1 / denom
    h0 = _gelu(_DOT(x, w1_ref[...]) + b1_ref[...])
    h1 = _gelu(_DOT(a_ref[...], w1_ref[...]) + b1_ref[...])
    hm_ref[...] = _bf(g0 * h0 + g1 * h1)
    g_ref[...] = jnp.concatenate([g0, g1], axis=1)


def _out_kernel(hm_ref, w2_ref, b2_ref, g_ref, o_ref):
    g = g_ref[...]
    gsum = g[:, 0:1] + g[:, 1:2]
    o_ref[...] = _DOT(hm_ref[...], w2_ref[...]) + gsum * b2_ref[...]


def kernel(x, proj_w1, proj_b1, proj_w2, proj_b2, query, ln_q_g, ln_q_b,
           ln_kv_g, ln_kv_b, in_proj_w, in_proj_b, out_proj_w, out_proj_b,
           w_gate, pos_embed):
    f32 = jnp.float32
    xf = x.reshape(TOKENS, MM)
    bq, bk, bv = (in_proj_b[:MM][None, :], in_proj_b[MM:2 * MM][None, :],
                  in_proj_b[2 * MM:][None, :])
    ln_q_g2, ln_q_b2 = ln_q_g[None, :], ln_q_b[None, :]
    ln_kv_g2, ln_kv_b2 = ln_kv_g[None, :], ln_kv_b[None, :]
    b1_2 = proj_b1[None, :]
    b2_2 = proj_b2[None, :]
    ob_2 = out_proj_b[None, :]
    wgt = w_gate.T
    bf16 = jnp.bfloat16

    qh = pl.pallas_call(
        _q_kernel,
        grid=(1,),
        in_specs=[
            pl.BlockSpec((N_Q, MM), lambda i: (0, 0)),
            pl.BlockSpec((N_Q, MM), lambda i: (0, 0)),
            pl.BlockSpec((1, MM), lambda i: (0, 0)),
            pl.BlockSpec((1, MM), lambda i: (0, 0)),
            pl.BlockSpec((MM, MM), lambda i: (0, 0)),
            pl.BlockSpec((1, MM), lambda i: (0, 0)),
        ],
        out_specs=pl.BlockSpec((N_Q, MM), lambda i: (0, 0)),
        out_shape=jax.ShapeDtypeStruct((N_Q, MM), f32),
    )(query, pos_embed, ln_q_g2, ln_q_b2, in_proj_w, bq)

    attn_out = pl.pallas_call(
        _resampler_kernel,
        grid=(BATCH,),
        in_specs=[
            pl.BlockSpec((N_Q, MM), lambda i: (0, 0)),
            pl.BlockSpec((N_Q, MM), lambda i: (i, 0)),
            pl.BlockSpec((N_Q, MM), lambda i: (0, 0)),
            pl.BlockSpec((1, MM), lambda i: (0, 0)),
            pl.BlockSpec((1, MM), lambda i: (0, 0)),
            pl.BlockSpec((MM, MM), lambda i: (1, 0)),
            pl.BlockSpec((MM, MM), lambda i: (2, 0)),
            pl.BlockSpec((1, MM), lambda i: (0, 0)),
            pl.BlockSpec((1, MM), lambda i: (0, 0)),
            pl.BlockSpec((MM, MM), lambda i: (0, 0)),
            pl.BlockSpec((1, MM), lambda i: (0, 0)),
        ],
        out_specs=pl.BlockSpec((N_Q, MM), lambda i: (i, 0)),
        out_shape=jax.ShapeDtypeStruct((TOKENS, MM), bf16),
        compiler_params=pltpu.CompilerParams(
            dimension_semantics=("parallel",)),
    )(qh, xf, pos_embed, ln_kv_g2, ln_kv_b2, in_proj_w, in_proj_w,
      bk, bv, out_proj_w, ob_2)

    MT = 256
    hm, gates = pl.pallas_call(
        _h_kernel,
        grid=(TOKENS // MT,),
        in_specs=[
            pl.BlockSpec((MT, MM), lambda i: (i, 0)),
            pl.BlockSpec((MT, MM), lambda i: (i, 0)),
            pl.BlockSpec((MM, HID), lambda i: (0, 0)),
            pl.BlockSpec((1, HID), lambda i: (0, 0)),
            pl.BlockSpec((2, MM), lambda i: (0, 0)),
        ],
        out_specs=[
            pl.BlockSpec((MT, HID), lambda i: (i, 0)),
            pl.BlockSpec((MT, 2), lambda i: (i, 0)),
        ],
        out_shape=[
            jax.ShapeDtypeStruct((TOKENS, HID), bf16),
            jax.ShapeDtypeStruct((TOKENS, 2), f32),
        ],
        compiler_params=pltpu.CompilerParams(
            dimension_semantics=("parallel",)),
    )(xf, attn_out, proj_w1, b1_2, wgt)

    out = pl.pallas_call(
        _out_kernel,
        grid=(TOKENS // MT,),
        in_specs=[
            pl.BlockSpec((MT, HID), lambda i: (i, 0)),
            pl.BlockSpec((HID, HID), lambda i: (0, 0)),
            pl.BlockSpec((1, HID), lambda i: (0, 0)),
            pl.BlockSpec((MT, 2), lambda i: (i, 0)),
        ],
        out_specs=pl.BlockSpec((MT, HID), lambda i: (i, 0)),
        out_shape=jax.ShapeDtypeStruct((TOKENS, HID), f32),
        compiler_params=pltpu.CompilerParams(
            dimension_semantics=("parallel",)),
    )(hm, proj_w2, b2_2, gates)

    return out.reshape(BATCH, N_Q, HID)


# speedup vs baseline: 1.2248x; 1.0360x over previous
import functools

import jax
import jax.numpy as jnp
from jax.experimental import pallas as pl
from jax.experimental.pallas import tpu as pltpu

MM = 1024
HID = 2048
N_Q = 576
HEADS = 8
HEAD_DIM = 128
BATCH = 4
TOKENS = BATCH * N_Q

def _bf(a):
    return a.astype(jnp.bfloat16)


def _DOT(a, b):
    return jnp.dot(a, b, preferred_element_type=jnp.float32)


def _dot_t(a, b):
    return jax.lax.dot_general(a, b, (((1,), (1,)), ((), ())),
                               preferred_element_type=jnp.float32)


def _ln(x, g, b, eps=1e-5):
    mu = jnp.mean(x, axis=-1, keepdims=True)
    var = jnp.mean((x - mu) ** 2, axis=-1, keepdims=True)
    return (x - mu) * jax.lax.rsqrt(var + eps) * g + b


def _gelu(h):
    return 0.5 * h * (1.0 + jax.lax.erf(h * 0.7071067811865476))


def _q_kernel(query_ref, pos_ref, g_ref, b_ref, wq_ref, bq_ref, qh_ref):
    qf = _ln(query_ref[...], g_ref[...], b_ref[...]) + pos_ref[...]
    qh_ref[...] = _dot_t(qf, wq_ref[...]) + bq_ref[...]


def _resampler_kernel(qh_ref, x_ref, pos_ref, g_ref, b_ref, wk_ref, wv_ref,
                      bk_ref, bv_ref, ow_ref, ob_ref, o_ref):
    kv = _ln(x_ref[...], g_ref[...], b_ref[...])
    kh = _dot_t(kv + pos_ref[...], wk_ref[...]) + bk_ref[...]
    vh = _dot_t(kv, wv_ref[...]) + bv_ref[...]
    qh = qh_ref[...]
    scale = 1.0 / (HEAD_DIM ** 0.5)
    outs = []
    for h in range(HEADS):
        sl = slice(h * HEAD_DIM, (h + 1) * HEAD_DIM)
        s = _dot_t(qh[:, sl], kh[:, sl]) * scale
        s = s - jnp.max(s, axis=-1, keepdims=True)
        e = jnp.exp(s)
        p = e / jnp.sum(e, axis=-1, keepdims=True)
        outs.append(_DOT(p, vh[:, sl]))
    o = jnp.concatenate(outs, axis=1)
    o_ref[...] = _bf(_dot_t(o, ow_ref[...]) + ob_ref[...])


def _mlp_kernel(x_ref, a_ref, w1_ref, b1_ref, w2_ref, b2_ref, wgt_ref, o_ref):
    x = x_ref[...]
    wgt = wgt_ref[...]
    l0 = jnp.sum(x * wgt[0:1, :], axis=1, keepdims=True)
    l1 = jnp.sum(x * wgt[1:2, :], axis=1, keepdims=True)
    m = jnp.maximum(l0, l1)
    e0 = jnp.exp(l0 - m)
    e1 = jnp.exp(l1 - m)
    s = e0 + e1
    p0 = e0 / s
    p1 = e1 / s
    denom = p0 + p1 + 1e-6
    g0 = p0 / denom
    g1 = p1 / denom
    h0 = _gelu(_DOT(x, w1_ref[...]) + b1_ref[...])
    h1 = _gelu(_DOT(a_ref[...], w1_ref[...]) + b1_ref[...])
    hm = g0 * h0 + g1 * h1
    o_ref[...] = _DOT(hm, w2_ref[...]) + (g0 + g1) * b2_ref[...]


def kernel(x, proj_w1, proj_b1, proj_w2, proj_b2, query, ln_q_g, ln_q_b,
           ln_kv_g, ln_kv_b, in_proj_w, in_proj_b, out_proj_w, out_proj_b,
           w_gate, pos_embed):
    f32 = jnp.float32
    xf = x.reshape(TOKENS, MM)
    bq, bk, bv = (in_proj_b[:MM][None, :], in_proj_b[MM:2 * MM][None, :],
                  in_proj_b[2 * MM:][None, :])
    ln_q_g2, ln_q_b2 = ln_q_g[None, :], ln_q_b[None, :]
    ln_kv_g2, ln_kv_b2 = ln_kv_g[None, :], ln_kv_b[None, :]
    b1_2 = proj_b1[None, :]
    b2_2 = proj_b2[None, :]
    ob_2 = out_proj_b[None, :]
    wgt = w_gate.T
    bf16 = jnp.bfloat16

    qh = pl.pallas_call(
        _q_kernel,
        grid=(1,),
        in_specs=[
            pl.BlockSpec((N_Q, MM), lambda i: (0, 0)),
            pl.BlockSpec((N_Q, MM), lambda i: (0, 0)),
            pl.BlockSpec((1, MM), lambda i: (0, 0)),
            pl.BlockSpec((1, MM), lambda i: (0, 0)),
            pl.BlockSpec((MM, MM), lambda i: (0, 0)),
            pl.BlockSpec((1, MM), lambda i: (0, 0)),
        ],
        out_specs=pl.BlockSpec((N_Q, MM), lambda i: (0, 0)),
        out_shape=jax.ShapeDtypeStruct((N_Q, MM), f32),
    )(query, pos_embed, ln_q_g2, ln_q_b2, in_proj_w, bq)

    attn_out = pl.pallas_call(
        _resampler_kernel,
        grid=(BATCH,),
        in_specs=[
            pl.BlockSpec((N_Q, MM), lambda i: (0, 0)),
            pl.BlockSpec((N_Q, MM), lambda i: (i, 0)),
            pl.BlockSpec((N_Q, MM), lambda i: (0, 0)),
            pl.BlockSpec((1, MM), lambda i: (0, 0)),
            pl.BlockSpec((1, MM), lambda i: (0, 0)),
            pl.BlockSpec((MM, MM), lambda i: (1, 0)),
            pl.BlockSpec((MM, MM), lambda i: (2, 0)),
            pl.BlockSpec((1, MM), lambda i: (0, 0)),
            pl.BlockSpec((1, MM), lambda i: (0, 0)),
            pl.BlockSpec((MM, MM), lambda i: (0, 0)),
            pl.BlockSpec((1, MM), lambda i: (0, 0)),
        ],
        out_specs=pl.BlockSpec((N_Q, MM), lambda i: (i, 0)),
        out_shape=jax.ShapeDtypeStruct((TOKENS, MM), bf16),
        compiler_params=pltpu.CompilerParams(
            dimension_semantics=("parallel",)),
    )(qh, xf, pos_embed, ln_kv_g2, ln_kv_b2, in_proj_w, in_proj_w,
      bk, bv, out_proj_w, ob_2)

    MT = 256
    out = pl.pallas_call(
        _mlp_kernel,
        grid=(TOKENS // MT,),
        in_specs=[
            pl.BlockSpec((MT, MM), lambda i: (i, 0)),
            pl.BlockSpec((MT, MM), lambda i: (i, 0)),
            pl.BlockSpec((MM, HID), lambda i: (0, 0)),
            pl.BlockSpec((1, HID), lambda i: (0, 0)),
            pl.BlockSpec((HID, HID), lambda i: (0, 0)),
            pl.BlockSpec((1, HID), lambda i: (0, 0)),
            pl.BlockSpec((2, MM), lambda i: (0, 0)),
        ],
        out_specs=pl.BlockSpec((MT, HID), lambda i: (i, 0)),
        out_shape=jax.ShapeDtypeStruct((TOKENS, HID), f32),
        compiler_params=pltpu.CompilerParams(
            dimension_semantics=("parallel",)),
    )(xf, attn_out, proj_w1, b1_2, proj_w2, b2_2, wgt)

    return out.reshape(BATCH, N_Q, HID)
